# Initial kernel scaffold; baseline (speedup 1.0000x reference)
import jax
import jax.numpy as jnp
from jax.experimental import pallas as pl


def kernel(x, edge_index, W1_l, W1_r, b1, W2_l, W2_r, b2):
    raise NotImplementedError("write your pallas kernel here")


# speedup vs baseline: 10.1883x; 10.1883x over previous
import functools

import jax
import jax.numpy as jnp
from jax import lax
from jax.experimental import pallas as pl
from jax.experimental.pallas import tpu as pltpu
from jax.experimental.pallas import tpu_sc as plsc

NC = 2
NS = 16
NW = NC * NS
CH = 128
F32 = jnp.float32


def _flat_worker_id():
    return lax.axis_index("s") * NC + lax.axis_index("c")


def _zero_zbuf(zbuf, zr, w):
    def body(i, carry):
        for c0 in range(0, w, 16):
            zbuf[i, c0:c0 + 16] = jnp.zeros((16,), F32)
        return carry
    lax.fori_loop(0, zr, body, 0)


def _zero_acc_slice(zbuf, acc, base, rows, zr):
    n_full, rem = rows // zr, rows % zr
    for t in range(n_full):
        pltpu.sync_copy(zbuf, acc.at[pl.ds(base + t * zr, zr)])
    if rem:
        pltpu.sync_copy(zbuf.at[0:rem], acc.at[pl.ds(base + n_full * zr, rem)])


def _write_partial(acc, out_h, c, s, n):
    per_t = (((n + NS - 1) // NS) + 7) // 8 * 8
    last = n - (NS - 1) * per_t

    @pl.when(s < NS - 1)
    def _():
        base = s * per_t
        pltpu.sync_copy(acc.at[pl.ds(base, per_t)],
                        out_h.at[c].at[pl.ds(base, per_t)])

    @pl.when(s == NS - 1)
    def _():
        base = (NS - 1) * per_t
        pltpu.sync_copy(acc.at[pl.ds(base, last)],
                        out_h.at[c].at[pl.ds(base, last)])


def _idx_block(k):
    for d in range(16, 0, -1):
        if k % d == 0:
            return d
    return 1


def _sc_segment_sum(tab, src_r, dst_r, n_nodes):
    n, w = tab.shape
    k = src_r.shape[1]
    ib = _idx_block(k)
    nb = k // ib
    nacc = ((n + 1 + 127) // 128) * 128
    rows_per_tile_z = nacc // NS
    zr = 128
    mesh = plsc.VectorSubcoreMesh(core_axis_name="c", subcore_axis_name="s", num_cores=NC, num_subcores=NS)

    @functools.partial(
        pl.kernel,
        out_type=jax.ShapeDtypeStruct((NC, n, w), F32),
        mesh=mesh,
        compiler_params=pltpu.CompilerParams(use_tc_tiling_on_sc=False),
        scratch_types=[
            pltpu.VMEM((ib, CH), jnp.int32),
            pltpu.VMEM((ib, CH), jnp.int32),
            pltpu.VMEM((CH, w), F32),
            pltpu.VMEM((zr, w), F32),
            pltpu.VMEM_SHARED((nacc, w), F32),
            pltpu.SemaphoreType.DMA,
        ],
    )
    def k_fn(tab_h, src_h, dst_h, out_h, src_v, dst_v, rows_v, zbuf, acc, sem):
        c = lax.axis_index("c")
        s = lax.axis_index("s")
        wid = _flat_worker_id()
        _zero_zbuf(zbuf, zr, w)
        _zero_acc_slice(zbuf, acc, s * rows_per_tile_z, rows_per_tile_z, zr)
        plsc.subcore_barrier()

        def blk_body(b, carry):
            pltpu.sync_copy(src_h.at[wid].at[pl.ds(b * ib, ib)], src_v)
            pltpu.sync_copy(dst_h.at[wid].at[pl.ds(b * ib, ib)], dst_v)

            def body(j, carry2):
                pltpu.async_copy(tab_h.at[src_v.at[j]], rows_v, sem).wait()
                pltpu.sync_copy(rows_v, acc.at[dst_v.at[j]], add=True)
                return carry2
            lax.fori_loop(0, ib, body, 0)
            return carry
        lax.fori_loop(0, nb, blk_body, 0)
        plsc.subcore_barrier()
        _write_partial(acc, out_h, c, s, n)

    return k_fn(tab, src_r, dst_r)


def _sc_counts(dst_r, n_nodes):
    n = n_nodes
    w = 16
    k = dst_r.shape[1]
    ib = _idx_block(k)
    nb = k // ib
    nacc = ((n + 1 + 127) // 128) * 128
    rows_per_tile_z = nacc // NS
    zr = 128
    mesh = plsc.VectorSubcoreMesh(core_axis_name="c", subcore_axis_name="s", num_cores=NC, num_subcores=NS)

    @functools.partial(
        pl.kernel,
        out_type=jax.ShapeDtypeStruct((NC, n, w), F32),
        mesh=mesh,
        compiler_params=pltpu.CompilerParams(use_tc_tiling_on_sc=False),
        scratch_types=[
            pltpu.VMEM((ib, CH), jnp.int32),
            pltpu.VMEM((CH, w), F32),
            pltpu.VMEM((zr, w), F32),
            pltpu.VMEM_SHARED((nacc, w), F32),
            pltpu.SemaphoreType.DMA,
        ],
    )
    def k_fn(dst_h, out_h, dst_v, ones_v, zbuf, acc, sem):
        c = lax.axis_index("c")
        s = lax.axis_index("s")
        wid = _flat_worker_id()

        def fill_ones(i, carry):
            ones_v[i, 0:16] = jnp.ones((16,), F32)
            return carry
        lax.fori_loop(0, CH, fill_ones, 0)
        _zero_zbuf(zbuf, zr, w)
        _zero_acc_slice(zbuf, acc, s * rows_per_tile_z, rows_per_tile_z, zr)
        plsc.subcore_barrier()

        def blk_body(b, carry):
            pltpu.sync_copy(dst_h.at[wid].at[pl.ds(b * ib, ib)], dst_v)

            def body(j, carry2):
                pltpu.sync_copy(ones_v, acc.at[dst_v.at[j]], add=True)
                return carry2
            lax.fori_loop(0, ib, body, 0)
            return carry
        lax.fori_loop(0, nb, blk_body, 0)
        plsc.subcore_barrier()
        _write_partial(acc, out_h, c, s, n)

    return k_fn(dst_r)


def _tc_pre(x, w1_l, w1_r, b1, blk):
    n, din = x.shape
    dh = w1_l.shape[1]

    def body(x_ref, wl_ref, wr_ref, b_ref, y_ref, r_ref):
        xb = x_ref[...]
        y_ref[...] = jnp.dot(xb, wl_ref[...], preferred_element_type=F32)
        r_ref[...] = jnp.dot(xb, wr_ref[...], preferred_element_type=F32) + b_ref[...]

    return pl.pallas_call(
        body,
        grid=(n // blk,),
        in_specs=[
            pl.BlockSpec((blk, din), lambda i: (i, 0)),
            pl.BlockSpec((din, dh), lambda i: (0, 0)),
            pl.BlockSpec((din, dh), lambda i: (0, 0)),
            pl.BlockSpec((1, dh), lambda i: (0, 0)),
        ],
        out_specs=[
            pl.BlockSpec((blk, dh), lambda i: (i, 0)),
            pl.BlockSpec((blk, dh), lambda i: (i, 0)),
        ],
        out_shape=[
            jax.ShapeDtypeStruct((n, dh), F32),
            jax.ShapeDtypeStruct((n, dh), F32),
        ],
    )(x, w1_l, w1_r, b1.reshape(1, dh))


def _tc_mid(sum1, cnt, r1, blk):
    n, dh = r1.shape

    def body(s_ref, c_ref, r_ref, h_ref):
        ssum = s_ref[0] + s_ref[1]
        c = c_ref[0, :, 0:1] + c_ref[1, :, 0:1]
        rcp = 1.0 / jnp.maximum(c, 1.0)
        h_ref[...] = jnp.maximum(ssum * rcp + r_ref[...], 0.0)

    return pl.pallas_call(
        body,
        grid=(n // blk,),
        in_specs=[
            pl.BlockSpec((NC, blk, dh), lambda i: (0, i, 0)),
            pl.BlockSpec((NC, blk, 16), lambda i: (0, i, 0)),
            pl.BlockSpec((blk, dh), lambda i: (i, 0)),
        ],
        out_specs=pl.BlockSpec((blk, dh), lambda i: (i, 0)),
        out_shape=jax.ShapeDtypeStruct((n, dh), F32),
    )(sum1, cnt, r1)


def _tc_post(sum2, cnt, h, w2_l, w2_r, b2, blk):
    n, dh = h.shape
    dout = w2_l.shape[1]

    def body(s_ref, c_ref, h_ref, wl_ref, wr_ref, b_ref, o_ref):
        ssum = s_ref[0] + s_ref[1]
        c = c_ref[0, :, 0:1] + c_ref[1, :, 0:1]
        mean2 = ssum * (1.0 / jnp.maximum(c, 1.0))
        o = (jnp.dot(mean2, wl_ref[...], preferred_element_type=F32)
             + b_ref[...]
             + jnp.dot(h_ref[...], wr_ref[...], preferred_element_type=F32))
        o_ref[...] = jnp.maximum(o, 0.0)

    return pl.pallas_call(
        body,
        grid=(n // blk,),
        in_specs=[
            pl.BlockSpec((NC, blk, dh), lambda i: (0, i, 0)),
            pl.BlockSpec((NC, blk, 16), lambda i: (0, i, 0)),
            pl.BlockSpec((blk, dh), lambda i: (i, 0)),
            pl.BlockSpec((dh, dout), lambda i: (0, 0)),
            pl.BlockSpec((dh, dout), lambda i: (0, 0)),
            pl.BlockSpec((1, dout), lambda i: (0, 0)),
        ],
        out_specs=pl.BlockSpec((blk, dout), lambda i: (i, 0)),
        out_shape=jax.ShapeDtypeStruct((n, dout), F32),
    )(sum2, cnt, h, w2_l, w2_r, b2.reshape(1, dout))


def kernel(x, edge_index, W1_l, W1_r, b1, W2_l, W2_r, b2):
    n, din = x.shape
    e = edge_index.shape[1]
    src = edge_index[0].astype(jnp.int32)
    dst = edge_index[1].astype(jnp.int32)

    k = -(-e // (NW * CH))
    e_pad = NW * k * CH
    pad = e_pad - e
    src_p = jnp.concatenate([src, jnp.zeros((pad,), jnp.int32)])
    dst_p = jnp.concatenate([dst, jnp.full((pad,), n, jnp.int32)])
    src_r = src_p.reshape(NW, k, CH)
    dst_r = dst_p.reshape(NW, k, CH)

    blk = 2000
    y1, r1 = _tc_pre(x, W1_l, W1_r, b1, blk)
    cnt = _sc_counts(dst_r, n)
    sum1 = _sc_segment_sum(y1, src_r, dst_r, n)
    h = _tc_mid(sum1, cnt, r1, blk)
    sum2 = _sc_segment_sum(h, src_r, dst_r, n)
    return _tc_post(sum2, cnt, h, W2_l, W2_r, b2, blk)


# speedup vs baseline: 11.4062x; 1.1195x over previous
import functools

import jax
import jax.numpy as jnp
from jax import lax
from jax.experimental import pallas as pl
from jax.experimental.pallas import tpu as pltpu
from jax.experimental.pallas import tpu_sc as plsc

NC = 2
NS = 16
NW = NC * NS
CH = 128
F32 = jnp.float32


def _flat_worker_id():
    return lax.axis_index("s") * NC + lax.axis_index("c")


def _zero_zbuf(zbuf, zr, w):
    def body(i, carry):
        for c0 in range(0, w, 16):
            zbuf[i, c0:c0 + 16] = jnp.zeros((16,), F32)
        return carry
    lax.fori_loop(0, zr, body, 0)


def _zero_acc_slice(zbuf, acc, base, rows, zr):
    n_full, rem = rows // zr, rows % zr
    for t in range(n_full):
        pltpu.sync_copy(zbuf, acc.at[pl.ds(base + t * zr, zr)])
    if rem:
        pltpu.sync_copy(zbuf.at[0:rem], acc.at[pl.ds(base + n_full * zr, rem)])


def _write_partial(acc, out_h, c, s, n):
    per_t = (((n + NS - 1) // NS) + 7) // 8 * 8
    last = n - (NS - 1) * per_t

    @pl.when(s < NS - 1)
    def _():
        base = s * per_t
        pltpu.sync_copy(acc.at[pl.ds(base, per_t)],
                        out_h.at[c].at[pl.ds(base, per_t)])

    @pl.when(s == NS - 1)
    def _():
        base = (NS - 1) * per_t
        pltpu.sync_copy(acc.at[pl.ds(base, last)],
                        out_h.at[c].at[pl.ds(base, last)])


def _idx_block(k, cap):
    for d in range(cap, 1, -1):
        if d % 2 == 0 and k % d == 0:
            return d
    return 0


def _zero_acc_async(zbuf, acc, base, rows, zr, zsem):
    n_full, rem = rows // zr, rows % zr
    for t in range(n_full):
        pltpu.async_copy(zbuf, acc.at[pl.ds(base + t * zr, zr)], zsem)
    if rem:
        pltpu.async_copy(zbuf.at[0:rem], acc.at[pl.ds(base + n_full * zr, rem)], zsem)
    for t in range(n_full):
        pltpu.make_async_copy(zbuf, acc.at[pl.ds(base + t * zr, zr)], zsem).wait()
    if rem:
        pltpu.make_async_copy(zbuf.at[0:rem], acc.at[pl.ds(base + n_full * zr, rem)], zsem).wait()


def _sc_segment_sum(tab, src_r, dst_r, n_nodes):
    n, w = tab.shape
    k = src_r.shape[1]
    ib = _idx_block(k, 28)
    nb = k // ib
    npairs = ib // 2
    nacc = ((n + 1 + 127) // 128) * 128
    rows_per_tile_z = nacc // NS
    zr = 128
    mesh = plsc.VectorSubcoreMesh(core_axis_name="c", subcore_axis_name="s", num_cores=NC, num_subcores=NS)

    @functools.partial(
        pl.kernel,
        out_type=jax.ShapeDtypeStruct((NC, n, w), F32),
        mesh=mesh,
        compiler_params=pltpu.CompilerParams(use_tc_tiling_on_sc=False),
        scratch_types=[
            pltpu.VMEM((ib, CH), jnp.int32),
            pltpu.VMEM((ib, CH), jnp.int32),
            pltpu.VMEM((CH, w), F32),
            pltpu.VMEM((CH, w), F32),
            pltpu.VMEM((zr, w), F32),
            pltpu.VMEM_SHARED((nacc, w), F32),
            pltpu.SemaphoreType.DMA,
            pltpu.SemaphoreType.DMA,
            pltpu.SemaphoreType.DMA,
            pltpu.SemaphoreType.DMA,
            pltpu.SemaphoreType.DMA,
        ],
    )
    def k_fn(tab_h, src_h, dst_h, out_h, src_v, dst_v, rows_a, rows_b, zbuf,
             acc, gsa, gsb, ssa, ssb, zsem):
        c = lax.axis_index("c")
        s = lax.axis_index("s")
        wid = _flat_worker_id()
        _zero_zbuf(zbuf, zr, w)
        _zero_acc_async(zbuf, acc, s * rows_per_tile_z, rows_per_tile_z, zr, zsem)
        plsc.subcore_barrier()

        def blk_body(b, carry):
            base = b * ib
            pltpu.sync_copy(src_h.at[wid].at[pl.ds(base, ib)], src_v)
            pltpu.sync_copy(dst_h.at[wid].at[pl.ds(base, ib)], dst_v)
            pltpu.async_copy(tab_h.at[src_v.at[0]], rows_a, gsa)

            def pair(p, carry2):
                j0 = 2 * p
                pltpu.make_async_copy(tab_h.at[src_v.at[j0]], rows_a, gsa).wait()

                @pl.when(p > 0)
                def _():
                    pltpu.make_async_copy(rows_b, acc.at[dst_v.at[0]], ssb).wait()
                pltpu.async_copy(tab_h.at[src_v.at[j0 + 1]], rows_b, gsb)
                pltpu.async_copy(rows_a, acc.at[dst_v.at[j0]], ssa, add=True)
                pltpu.make_async_copy(tab_h.at[src_v.at[j0 + 1]], rows_b, gsb).wait()
                pltpu.make_async_copy(rows_a, acc.at[dst_v.at[j0]], ssa).wait()

                @pl.when(p < npairs - 1)
                def _():
                    pltpu.async_copy(tab_h.at[src_v.at[j0 + 2]], rows_a, gsa)
                pltpu.async_copy(rows_b, acc.at[dst_v.at[j0 + 1]], ssb, add=True)
                return carry2
            lax.fori_loop(0, npairs, pair, 0)
            pltpu.make_async_copy(rows_b, acc.at[dst_v.at[0]], ssb).wait()
            return carry
        lax.fori_loop(0, nb, blk_body, 0)
        plsc.subcore_barrier()
        _write_partial(acc, out_h, c, s, n)

    return k_fn(tab, src_r, dst_r)


def _sc_counts(dst_r, n_nodes):
    n = n_nodes
    w = 16
    k = dst_r.shape[1]
    ib = _idx_block(k, 16)
    nb = k // ib
    nacc = ((n + 1 + 127) // 128) * 128
    rows_per_tile_z = nacc // NS
    zr = 128
    mesh = plsc.VectorSubcoreMesh(core_axis_name="c", subcore_axis_name="s", num_cores=NC, num_subcores=NS)

    @functools.partial(
        pl.kernel,
        out_type=jax.ShapeDtypeStruct((NC, n, w), F32),
        mesh=mesh,
        compiler_params=pltpu.CompilerParams(use_tc_tiling_on_sc=False),
        scratch_types=[
            pltpu.VMEM((ib, CH), jnp.int32),
            pltpu.VMEM((CH, w), F32),
            pltpu.VMEM((zr, w), F32),
            pltpu.VMEM_SHARED((nacc, w), F32),
            pltpu.SemaphoreType.DMA,
        ],
    )
    def k_fn(dst_h, out_h, dst_v, ones_v, zbuf, acc, sem):
        c = lax.axis_index("c")
        s = lax.axis_index("s")
        wid = _flat_worker_id()

        def fill_ones(i, carry):
            ones_v[i, 0:16] = jnp.ones((16,), F32)
            return carry
        lax.fori_loop(0, CH, fill_ones, 0)
        _zero_zbuf(zbuf, zr, w)
        _zero_acc_slice(zbuf, acc, s * rows_per_tile_z, rows_per_tile_z, zr)
        plsc.subcore_barrier()

        def blk_body(b, carry):
            pltpu.sync_copy(dst_h.at[wid].at[pl.ds(b * ib, ib)], dst_v)

            def body(j, carry2):
                pltpu.sync_copy(ones_v, acc.at[dst_v.at[j]], add=True)
                return carry2
            lax.fori_loop(0, ib, body, 0)
            return carry
        lax.fori_loop(0, nb, blk_body, 0)
        plsc.subcore_barrier()
        _write_partial(acc, out_h, c, s, n)

    return k_fn(dst_r)


def _tc_pre(x, w1_l, w1_r, b1, blk):
    n, din = x.shape
    dh = w1_l.shape[1]

    def body(x_ref, wl_ref, wr_ref, b_ref, y_ref, r_ref):
        xb = x_ref[...]
        y_ref[...] = jnp.dot(xb, wl_ref[...], preferred_element_type=F32)
        r_ref[...] = jnp.dot(xb, wr_ref[...], preferred_element_type=F32) + b_ref[...]

    return pl.pallas_call(
        body,
        grid=(n // blk,),
        in_specs=[
            pl.BlockSpec((blk, din), lambda i: (i, 0)),
            pl.BlockSpec((din, dh), lambda i: (0, 0)),
            pl.BlockSpec((din, dh), lambda i: (0, 0)),
            pl.BlockSpec((1, dh), lambda i: (0, 0)),
        ],
        out_specs=[
            pl.BlockSpec((blk, dh), lambda i: (i, 0)),
            pl.BlockSpec((blk, dh), lambda i: (i, 0)),
        ],
        out_shape=[
            jax.ShapeDtypeStruct((n, dh), F32),
            jax.ShapeDtypeStruct((n, dh), F32),
        ],
    )(x, w1_l, w1_r, b1.reshape(1, dh))


def _tc_mid(sum1, cnt, r1, blk):
    n, dh = r1.shape

    def body(s_ref, c_ref, r_ref, h_ref):
        ssum = s_ref[0] + s_ref[1]
        c = c_ref[0, :, 0:1] + c_ref[1, :, 0:1]
        rcp = 1.0 / jnp.maximum(c, 1.0)
        h_ref[...] = jnp.maximum(ssum * rcp + r_ref[...], 0.0)

    return pl.pallas_call(
        body,
        grid=(n // blk,),
        in_specs=[
            pl.BlockSpec((NC, blk, dh), lambda i: (0, i, 0)),
            pl.BlockSpec((NC, blk, 16), lambda i: (0, i, 0)),
            pl.BlockSpec((blk, dh), lambda i: (i, 0)),
        ],
        out_specs=pl.BlockSpec((blk, dh), lambda i: (i, 0)),
        out_shape=jax.ShapeDtypeStruct((n, dh), F32),
    )(sum1, cnt, r1)


def _tc_post(sum2, cnt, h, w2_l, w2_r, b2, blk):
    n, dh = h.shape
    dout = w2_l.shape[1]

    def body(s_ref, c_ref, h_ref, wl_ref, wr_ref, b_ref, o_ref):
        ssum = s_ref[0] + s_ref[1]
        c = c_ref[0, :, 0:1] + c_ref[1, :, 0:1]
        mean2 = ssum * (1.0 / jnp.maximum(c, 1.0))
        o = (jnp.dot(mean2, wl_ref[...], preferred_element_type=F32)
             + b_ref[...]
             + jnp.dot(h_ref[...], wr_ref[...], preferred_element_type=F32))
        o_ref[...] = jnp.maximum(o, 0.0)

    return pl.pallas_call(
        body,
        grid=(n // blk,),
        in_specs=[
            pl.BlockSpec((NC, blk, dh), lambda i: (0, i, 0)),
            pl.BlockSpec((NC, blk, 16), lambda i: (0, i, 0)),
            pl.BlockSpec((blk, dh), lambda i: (i, 0)),
            pl.BlockSpec((dh, dout), lambda i: (0, 0)),
            pl.BlockSpec((dh, dout), lambda i: (0, 0)),
            pl.BlockSpec((1, dout), lambda i: (0, 0)),
        ],
        out_specs=pl.BlockSpec((blk, dout), lambda i: (i, 0)),
        out_shape=jax.ShapeDtypeStruct((n, dout), F32),
    )(sum2, cnt, h, w2_l, w2_r, b2.reshape(1, dout))


def kernel(x, edge_index, W1_l, W1_r, b1, W2_l, W2_r, b2):
    n, din = x.shape
    e = edge_index.shape[1]
    src = edge_index[0].astype(jnp.int32)
    dst = edge_index[1].astype(jnp.int32)

    k = -(-e // (NW * CH))
    e_pad = NW * k * CH
    pad = e_pad - e
    src_p = jnp.concatenate([src, jnp.zeros((pad,), jnp.int32)])
    dst_p = jnp.concatenate([dst, jnp.full((pad,), n, jnp.int32)])
    src_r = src_p.reshape(NW, k, CH)
    dst_r = dst_p.reshape(NW, k, CH)

    blk = 2000
    y1, r1 = _tc_pre(x, W1_l, W1_r, b1, blk)
    cnt = _sc_counts(dst_r, n)
    sum1 = _sc_segment_sum(y1, src_r, dst_r, n)
    h = _tc_mid(sum1, cnt, r1, blk)
    sum2 = _sc_segment_sum(h, src_r, dst_r, n)
    return _tc_post(sum2, cnt, h, W2_l, W2_r, b2, blk)


# speedup vs baseline: 14.4002x; 1.2625x over previous
import functools

import jax
import jax.numpy as jnp
from jax import lax
from jax.experimental import pallas as pl
from jax.experimental.pallas import tpu as pltpu
from jax.experimental.pallas import tpu_sc as plsc

NC = 2
NS = 16
NW = NC * NS
CH = 128
F32 = jnp.float32


def _flat_worker_id():
    return lax.axis_index("s") * NC + lax.axis_index("c")


def _zero_zbuf(zbuf, zr, w):
    def body(i, carry):
        for c0 in range(0, w, 16):
            zbuf[i, c0:c0 + 16] = jnp.zeros((16,), F32)
        return carry
    lax.fori_loop(0, zr, body, 0)


def _zero_acc_async(zbuf, acc, base, rows, zr, zsem):
    n_full, rem = rows // zr, rows % zr
    for t in range(n_full):
        pltpu.async_copy(zbuf, acc.at[pl.ds(base + t * zr, zr)], zsem)
    if rem:
        pltpu.async_copy(zbuf.at[0:rem], acc.at[pl.ds(base + n_full * zr, rem)], zsem)
    for t in range(n_full):
        pltpu.make_async_copy(zbuf, acc.at[pl.ds(base + t * zr, zr)], zsem).wait()
    if rem:
        pltpu.make_async_copy(zbuf.at[0:rem], acc.at[pl.ds(base + n_full * zr, rem)], zsem).wait()


def _write_partial(acc, out_h, c, s, n):
    per_t = (((n + NS - 1) // NS) + 7) // 8 * 8
    last = n - (NS - 1) * per_t

    @pl.when(s < NS - 1)
    def _():
        base = s * per_t
        pltpu.sync_copy(acc.at[pl.ds(base, per_t)],
                        out_h.at[c].at[pl.ds(base, per_t)])

    @pl.when(s == NS - 1)
    def _():
        base = (NS - 1) * per_t
        pltpu.sync_copy(acc.at[pl.ds(base, last)],
                        out_h.at[c].at[pl.ds(base, last)])


def _idx_block(k, cap):
    for d in range(cap, 1, -1):
        if d % 2 == 0 and k % d == 0:
            return d
    return 0


def _tc_repack_edges(e3, n_nodes, rows_out):
    _, chunks, _ = e3.shape
    blk = rows_out // 8

    def body(e_ref, s_ref, d_ref):
        i = pl.program_id(0)
        row = lax.broadcasted_iota(jnp.int32, (blk, CH), 0) + i * blk
        valid = row < chunks
        s_ref[...] = jnp.where(valid, e_ref[0], 0)
        d_ref[...] = jnp.where(valid, e_ref[1], n_nodes)

    return pl.pallas_call(
        body,
        grid=(8,),
        in_specs=[pl.BlockSpec((2, blk, CH), lambda i: (0, i, 0))],
        out_specs=[
            pl.BlockSpec((blk, CH), lambda i: (i, 0)),
            pl.BlockSpec((blk, CH), lambda i: (i, 0)),
        ],
        out_shape=[
            jax.ShapeDtypeStruct((rows_out, CH), jnp.int32),
            jax.ShapeDtypeStruct((rows_out, CH), jnp.int32),
        ],
    )(e3)


def _sc_segment_sum(tab, src_p, dst_p, n_nodes):
    n, w = tab.shape
    kw = src_p.shape[0] // NW
    ib = _idx_block(kw, 28)
    nb = kw // ib
    npairs = ib // 2
    nacc = ((n + 1 + 127) // 128) * 128
    rows_per_tile_z = nacc // NS
    zr = 128
    mesh = plsc.VectorSubcoreMesh(core_axis_name="c", subcore_axis_name="s",
                                  num_cores=NC, num_subcores=NS)

    @functools.partial(
        pl.kernel,
        out_type=jax.ShapeDtypeStruct((NC, n, w), F32),
        mesh=mesh,
        compiler_params=pltpu.CompilerParams(use_tc_tiling_on_sc=False),
        scratch_types=[
            pltpu.VMEM((ib, CH), jnp.int32),
            pltpu.VMEM((ib, CH), jnp.int32),
            pltpu.VMEM((CH, w), F32),
            pltpu.VMEM((CH, w), F32),
            pltpu.VMEM((zr, w), F32),
            pltpu.VMEM_SHARED((nacc, w), F32),
            pltpu.SemaphoreType.DMA,
            pltpu.SemaphoreType.DMA,
            pltpu.SemaphoreType.DMA,
            pltpu.SemaphoreType.DMA,
            pltpu.SemaphoreType.DMA,
        ],
    )
    def k_fn(tab_h, src_h, dst_h, out_h, src_v, dst_v, rows_a, rows_b, zbuf,
             acc, gsa, gsb, ssa, ssb, zsem):
        c = lax.axis_index("c")
        s = lax.axis_index("s")
        wid = _flat_worker_id()
        _zero_zbuf(zbuf, zr, w)
        _zero_acc_async(zbuf, acc, s * rows_per_tile_z, rows_per_tile_z, zr, zsem)
        plsc.subcore_barrier()

        def blk_body(b, carry):
            base = wid * kw + b * ib
            pltpu.sync_copy(src_h.at[pl.ds(base, ib)], src_v)
            pltpu.sync_copy(dst_h.at[pl.ds(base, ib)], dst_v)
            pltpu.async_copy(tab_h.at[src_v.at[0]], rows_a, gsa)

            def pair(p, carry2):
                j0 = 2 * p
                pltpu.make_async_copy(tab_h.at[src_v.at[j0]], rows_a, gsa).wait()

                @pl.when(p > 0)
                def _():
                    pltpu.make_async_copy(rows_b, acc.at[dst_v.at[0]], ssb).wait()
                pltpu.async_copy(tab_h.at[src_v.at[j0 + 1]], rows_b, gsb)
                pltpu.async_copy(rows_a, acc.at[dst_v.at[j0]], ssa, add=True)
                pltpu.make_async_copy(tab_h.at[src_v.at[j0 + 1]], rows_b, gsb).wait()
                pltpu.make_async_copy(rows_a, acc.at[dst_v.at[j0]], ssa).wait()

                @pl.when(p < npairs - 1)
                def _():
                    pltpu.async_copy(tab_h.at[src_v.at[j0 + 2]], rows_a, gsa)
                pltpu.async_copy(rows_b, acc.at[dst_v.at[j0 + 1]], ssb, add=True)
                return carry2
            lax.fori_loop(0, npairs, pair, 0)
            pltpu.make_async_copy(rows_b, acc.at[dst_v.at[0]], ssb).wait()
            return carry
        lax.fori_loop(0, nb, blk_body, 0)
        plsc.subcore_barrier()
        _write_partial(acc, out_h, c, s, n)

    return k_fn(tab, src_p, dst_p)


def _sc_counts(dst_p, n_nodes):
    n = n_nodes
    w = 32
    kw = dst_p.shape[0] // NW
    ib = _idx_block(kw, 16)
    nb = kw // ib
    nacc = ((n + 1 + 127) // 128) * 128
    rows_per_tile_z = nacc // NS
    zr = 128
    mesh = plsc.VectorSubcoreMesh(core_axis_name="c", subcore_axis_name="s",
                                  num_cores=NC, num_subcores=NS)

    @functools.partial(
        pl.kernel,
        out_type=jax.ShapeDtypeStruct((NC, n, w), F32),
        mesh=mesh,
        compiler_params=pltpu.CompilerParams(use_tc_tiling_on_sc=False),
        scratch_types=[
            pltpu.VMEM((ib, CH), jnp.int32),
            pltpu.VMEM((CH, w), F32),
            pltpu.VMEM((zr, w), F32),
            pltpu.VMEM_SHARED((nacc, w), F32),
            pltpu.SemaphoreType.DMA,
            pltpu.SemaphoreType.DMA,
        ],
    )
    def k_fn(dst_h, out_h, dst_v, ones_v, zbuf, acc, sem, zsem):
        c = lax.axis_index("c")
        s = lax.axis_index("s")
        wid = _flat_worker_id()

        def fill_ones(i, carry):
            for c0 in range(0, w, 16):
                ones_v[i, c0:c0 + 16] = jnp.ones((16,), F32)
            return carry
        lax.fori_loop(0, CH, fill_ones, 0)
        _zero_zbuf(zbuf, zr, w)
        _zero_acc_async(zbuf, acc, s * rows_per_tile_z, rows_per_tile_z, zr, zsem)
        plsc.subcore_barrier()

        def blk_body(b, carry):
            base = wid * kw + b * ib
            pltpu.sync_copy(dst_h.at[pl.ds(base, ib)], dst_v)

            def body(j, carry2):
                pltpu.sync_copy(ones_v, acc.at[dst_v.at[j]], add=True)
                return carry2
            lax.fori_loop(0, ib, body, 0)
            return carry
        lax.fori_loop(0, nb, blk_body, 0)
        plsc.subcore_barrier()
        _write_partial(acc, out_h, c, s, n)

    return k_fn(dst_p)


def _tc_pre(x4, w4l, w4r, b4, blk4):
    n4, din4 = x4.shape

    def body(x_ref, wl_ref, wr_ref, b_ref, y_ref, r_ref):
        xb = x_ref[...]
        y_ref[...] = jnp.dot(xb, wl_ref[...], preferred_element_type=F32)
        r_ref[...] = jnp.dot(xb, wr_ref[...], preferred_element_type=F32) + b_ref[...]

    return pl.pallas_call(
        body,
        grid=(n4 // blk4,),
        in_specs=[
            pl.BlockSpec((blk4, din4), lambda i: (i, 0)),
            pl.BlockSpec((din4, CH), lambda i: (0, 0)),
            pl.BlockSpec((din4, CH), lambda i: (0, 0)),
            pl.BlockSpec((1, CH), lambda i: (0, 0)),
        ],
        out_specs=[
            pl.BlockSpec((blk4, CH), lambda i: (i, 0)),
            pl.BlockSpec((blk4, CH), lambda i: (i, 0)),
        ],
        out_shape=[
            jax.ShapeDtypeStruct((n4, CH), F32),
            jax.ShapeDtypeStruct((n4, CH), F32),
        ],
    )(x4, w4l, w4r, b4)


def _tc_mid(sum1p, cntp, r1p, blk4):
    n4 = r1p.shape[0]

    def body(s_ref, c_ref, r_ref, h_ref):
        ssum = s_ref[0] + s_ref[1]
        cc = c_ref[0] + c_ref[1]
        rcp = 1.0 / jnp.maximum(cc, 1.0)
        h_ref[...] = jnp.maximum(ssum * rcp + r_ref[...], 0.0)

    return pl.pallas_call(
        body,
        grid=(n4 // blk4,),
        in_specs=[
            pl.BlockSpec((NC, blk4, CH), lambda i: (0, i, 0)),
            pl.BlockSpec((NC, blk4, CH), lambda i: (0, i, 0)),
            pl.BlockSpec((blk4, CH), lambda i: (i, 0)),
        ],
        out_specs=pl.BlockSpec((blk4, CH), lambda i: (i, 0)),
        out_shape=jax.ShapeDtypeStruct((n4, CH), F32),
    )(sum1p, cntp, r1p)


def _tc_post(sum2p, cntp, hp, w4l2, w4r2, b4out, blk4):
    n4 = hp.shape[0]
    dout4 = w4l2.shape[1]

    def body(s_ref, c_ref, h_ref, wl_ref, wr_ref, b_ref, o_ref):
        ssum = s_ref[0] + s_ref[1]
        cc = c_ref[0] + c_ref[1]
        mean2 = ssum * (1.0 / jnp.maximum(cc, 1.0))
        o = (jnp.dot(mean2, wl_ref[...], preferred_element_type=F32)
             + b_ref[...]
             + jnp.dot(h_ref[...], wr_ref[...], preferred_element_type=F32))
        o_ref[...] = jnp.maximum(o, 0.0)

    return pl.pallas_call(
        body,
        grid=(n4 // blk4,),
        in_specs=[
            pl.BlockSpec((NC, blk4, CH), lambda i: (0, i, 0)),
            pl.BlockSpec((NC, blk4, CH), lambda i: (0, i, 0)),
            pl.BlockSpec((blk4, CH), lambda i: (i, 0)),
            pl.BlockSpec((CH, dout4), lambda i: (0, 0)),
            pl.BlockSpec((CH, dout4), lambda i: (0, 0)),
            pl.BlockSpec((1, dout4), lambda i: (0, 0)),
        ],
        out_specs=pl.BlockSpec((blk4, dout4), lambda i: (i, 0)),
        out_shape=jax.ShapeDtypeStruct((n4, dout4), F32),
    )(sum2p, cntp, hp, w4l2, w4r2, b4out)


def kernel(x, edge_index, W1_l, W1_r, b1, W2_l, W2_r, b2):
    n, din = x.shape
    dh = W1_l.shape[1]
    dout = W2_l.shape[1]
    e = edge_index.shape[1]
    chunks = e // CH
    kw = -(-chunks // NW)
    rows_out = NW * kw

    e3 = edge_index.astype(jnp.int32).reshape(2, chunks, CH)
    src_p, dst_p = _tc_repack_edges(e3, n, rows_out)

    eye4 = jnp.eye(4, dtype=F32)
    x4 = x.reshape(n // 4, 4 * din)
    w4l = jnp.kron(eye4, W1_l)
    w4r = jnp.kron(eye4, W1_r)
    b4 = jnp.tile(b1, 4).reshape(1, CH)

    blk4 = n // 4
    y1p, r1p = _tc_pre(x4, w4l, w4r, b4, blk4)

    cnt = _sc_counts(dst_p, n)
    cntp = cnt.reshape(NC, n // 4, CH)
    sum1 = _sc_segment_sum(y1p.reshape(n, dh), src_p, dst_p, n)
    hp = _tc_mid(sum1.reshape(NC, n // 4, CH), cntp, r1p, blk4)

    sum2 = _sc_segment_sum(hp.reshape(n, dh), src_p, dst_p, n)
    w4l2 = jnp.kron(eye4, W2_l)
    w4r2 = jnp.kron(eye4, W2_r)
    b4out = jnp.tile(b2, 4).reshape(1, 4 * dout)
    outp = _tc_post(sum2.reshape(NC, n // 4, CH), cntp, hp,
                    w4l2, w4r2, b4out, blk4)
    return outp.reshape(n, dout)


# speedup vs baseline: 18.3885x; 1.2770x over previous
import functools

import jax
import jax.numpy as jnp
from jax import lax
from jax.experimental import pallas as pl
from jax.experimental.pallas import tpu as pltpu
from jax.experimental.pallas import tpu_sc as plsc

NC = 2
NS = 16
NW = NC * NS
CH = 128
F32 = jnp.float32


def _flat_worker_id():
    return lax.axis_index("s") * NC + lax.axis_index("c")


def _zero_zbuf(zbuf, zr, w):
    def body(i, carry):
        for c0 in range(0, w, 16):
            zbuf[i, c0:c0 + 16] = jnp.zeros((16,), F32)
        return carry
    lax.fori_loop(0, zr, body, 0)


def _zero_acc_async(zbuf, acc, base, rows, zr, zsem):
    n_full, rem = rows // zr, rows % zr
    for t in range(n_full):
        pltpu.async_copy(zbuf, acc.at[pl.ds(base + t * zr, zr)], zsem)
    if rem:
        pltpu.async_copy(zbuf.at[0:rem], acc.at[pl.ds(base + n_full * zr, rem)], zsem)
    for t in range(n_full):
        pltpu.make_async_copy(zbuf, acc.at[pl.ds(base + t * zr, zr)], zsem).wait()
    if rem:
        pltpu.make_async_copy(zbuf.at[0:rem], acc.at[pl.ds(base + n_full * zr, rem)], zsem).wait()


def _write_partial(acc, out_h, c, s, n):
    per_t = (((n + NS - 1) // NS) + 7) // 8 * 8
    last = n - (NS - 1) * per_t

    @pl.when(s < NS - 1)
    def _():
        base = s * per_t
        pltpu.sync_copy(acc.at[pl.ds(base, per_t)],
                        out_h.at[c].at[pl.ds(base, per_t)])

    @pl.when(s == NS - 1)
    def _():
        base = (NS - 1) * per_t
        pltpu.sync_copy(acc.at[pl.ds(base, last)],
                        out_h.at[c].at[pl.ds(base, last)])


def _idx_block(k, cap, mult=2):
    for d in range(cap, 1, -1):
        if d % mult == 0 and k % d == 0:
            return d
    return 0


def _tc_repack_edges(e3, n_nodes, rows_out):
    _, chunks, _ = e3.shape
    blk = rows_out // 8

    def body(e_ref, s_ref, d_ref):
        i = pl.program_id(0)
        row = lax.broadcasted_iota(jnp.int32, (blk, CH), 0) + i * blk
        valid = row < chunks
        s_ref[...] = jnp.where(valid, e_ref[0], 0)
        d_ref[...] = jnp.where(valid, e_ref[1], n_nodes)

    return pl.pallas_call(
        body,
        grid=(8,),
        in_specs=[pl.BlockSpec((2, blk, CH), lambda i: (0, i, 0))],
        out_specs=[
            pl.BlockSpec((blk, CH), lambda i: (i, 0)),
            pl.BlockSpec((blk, CH), lambda i: (i, 0)),
        ],
        out_shape=[
            jax.ShapeDtypeStruct((rows_out, CH), jnp.int32),
            jax.ShapeDtypeStruct((rows_out, CH), jnp.int32),
        ],
    )(e3)


def _sc_segment_sum(tab, src_p, dst_p, n_nodes):
    n, w = tab.shape
    kw = src_p.shape[0] // NW
    ib = _idx_block(kw, 28, mult=4)
    nb = kw // ib
    nq = ib // 4
    nacc = ((n + 1 + 127) // 128) * 128
    rows_per_tile_z = nacc // NS
    zr = 128
    mesh = plsc.VectorSubcoreMesh(core_axis_name="c", subcore_axis_name="s",
                                  num_cores=NC, num_subcores=NS)

    @functools.partial(
        pl.kernel,
        out_type=jax.ShapeDtypeStruct((NC, n, w), F32),
        mesh=mesh,
        compiler_params=pltpu.CompilerParams(use_tc_tiling_on_sc=False),
        scratch_types=[
            pltpu.VMEM((ib, CH), jnp.int32),
            pltpu.VMEM((ib, CH), jnp.int32),
            [pltpu.VMEM((CH, w), F32)] * 4,
            pltpu.VMEM((zr, w), F32),
            pltpu.VMEM_SHARED((nacc, w), F32),
            [pltpu.SemaphoreType.DMA] * 4,
            [pltpu.SemaphoreType.DMA] * 4,
            pltpu.SemaphoreType.DMA,
        ],
    )
    def k_fn(tab_h, src_h, dst_h, out_h, src_v, dst_v, rows, zbuf,
             acc, gs, ss, zsem):
        c = lax.axis_index("c")
        s = lax.axis_index("s")
        wid = _flat_worker_id()
        _zero_zbuf(zbuf, zr, w)
        _zero_acc_async(zbuf, acc, s * rows_per_tile_z, rows_per_tile_z, zr, zsem)
        plsc.subcore_barrier()

        def gath(j, r):
            pltpu.async_copy(tab_h.at[src_v.at[j]], rows[r], gs[r])

        def gath_wait(r):
            pltpu.make_async_copy(tab_h.at[src_v.at[0]], rows[r], gs[r]).wait()

        def scat(j, r):
            pltpu.async_copy(rows[r], acc.at[dst_v.at[j]], ss[r], add=True)

        def scat_wait(r):
            pltpu.make_async_copy(rows[r], acc.at[dst_v.at[0]], ss[r]).wait()

        def blk_body(b, carry):
            base = wid * kw + b * ib
            pltpu.sync_copy(src_h.at[pl.ds(base, ib)], src_v)
            pltpu.sync_copy(dst_h.at[pl.ds(base, ib)], dst_v)
            for r in range(3):
                gath(r, r)

            def quad(q, carry2):
                j = 4 * q
                gath_wait(0)
                scat(j, 0)
                gath_wait(1)

                @pl.when(q > 0)
                def _():
                    scat_wait(3)
                gath(j + 3, 3)
                scat(j + 1, 1)
                gath_wait(2)
                scat_wait(0)

                @pl.when(q < nq - 1)
                def _():
                    gath(j + 4, 0)
                scat(j + 2, 2)
                gath_wait(3)
                scat_wait(1)

                @pl.when(q < nq - 1)
                def _():
                    gath(j + 5, 1)
                scat(j + 3, 3)
                scat_wait(2)

                @pl.when(q < nq - 1)
                def _():
                    gath(j + 6, 2)
                return carry2
            lax.fori_loop(0, nq, quad, 0)
            scat_wait(3)
            return carry
        lax.fori_loop(0, nb, blk_body, 0)
        plsc.subcore_barrier()
        _write_partial(acc, out_h, c, s, n)

    return k_fn(tab, src_p, dst_p)


def _sc_counts(dst_p, n_nodes):
    n = n_nodes
    w = 32
    kw = dst_p.shape[0] // NW
    ib = _idx_block(kw, 16)
    nb = kw // ib
    nacc = ((n + 1 + 127) // 128) * 128
    rows_per_tile_z = nacc // NS
    zr = 128
    mesh = plsc.VectorSubcoreMesh(core_axis_name="c", subcore_axis_name="s",
                                  num_cores=NC, num_subcores=NS)

    @functools.partial(
        pl.kernel,
        out_type=jax.ShapeDtypeStruct((NC, n, w), F32),
        mesh=mesh,
        compiler_params=pltpu.CompilerParams(use_tc_tiling_on_sc=False),
        scratch_types=[
            pltpu.VMEM((ib, CH), jnp.int32),
            pltpu.VMEM((CH, w), F32),
            pltpu.VMEM((zr, w), F32),
            pltpu.VMEM_SHARED((nacc, w), F32),
            pltpu.SemaphoreType.DMA,
            pltpu.SemaphoreType.DMA,
        ],
    )
    def k_fn(dst_h, out_h, dst_v, ones_v, zbuf, acc, sem, zsem):
        c = lax.axis_index("c")
        s = lax.axis_index("s")
        wid = _flat_worker_id()

        def fill_ones(i, carry):
            for c0 in range(0, w, 16):
                ones_v[i, c0:c0 + 16] = jnp.ones((16,), F32)
            return carry
        lax.fori_loop(0, CH, fill_ones, 0)
        _zero_zbuf(zbuf, zr, w)
        _zero_acc_async(zbuf, acc, s * rows_per_tile_z, rows_per_tile_z, zr, zsem)
        plsc.subcore_barrier()

        def blk_body(b, carry):
            base = wid * kw + b * ib
            pltpu.sync_copy(dst_h.at[pl.ds(base, ib)], dst_v)

            def body(j, carry2):
                pltpu.sync_copy(ones_v, acc.at[dst_v.at[j]], add=True)
                return carry2
            lax.fori_loop(0, ib, body, 0)
            return carry
        lax.fori_loop(0, nb, blk_body, 0)
        plsc.subcore_barrier()
        _write_partial(acc, out_h, c, s, n)

    return k_fn(dst_p)


def _tc_pre(x4, w4l, w4r, b4, blk4):
    n4, din4 = x4.shape

    def body(x_ref, wl_ref, wr_ref, b_ref, y_ref, r_ref):
        xb = x_ref[...]
        y_ref[...] = jnp.dot(xb, wl_ref[...], preferred_element_type=F32)
        r_ref[...] = jnp.dot(xb, wr_ref[...], preferred_element_type=F32) + b_ref[...]

    return pl.pallas_call(
        body,
        grid=(n4 // blk4,),
        in_specs=[
            pl.BlockSpec((blk4, din4), lambda i: (i, 0)),
            pl.BlockSpec((din4, CH), lambda i: (0, 0)),
            pl.BlockSpec((din4, CH), lambda i: (0, 0)),
            pl.BlockSpec((1, CH), lambda i: (0, 0)),
        ],
        out_specs=[
            pl.BlockSpec((blk4, CH), lambda i: (i, 0)),
            pl.BlockSpec((blk4, CH), lambda i: (i, 0)),
        ],
        out_shape=[
            jax.ShapeDtypeStruct((n4, CH), F32),
            jax.ShapeDtypeStruct((n4, CH), F32),
        ],
    )(x4, w4l, w4r, b4)


def _tc_mid(sum1p, cntp, r1p, blk4):
    n4 = r1p.shape[0]

    def body(s_ref, c_ref, r_ref, h_ref):
        ssum = s_ref[0] + s_ref[1]
        cc = c_ref[0] + c_ref[1]
        rcp = 1.0 / jnp.maximum(cc, 1.0)
        h_ref[...] = jnp.maximum(ssum * rcp + r_ref[...], 0.0)

    return pl.pallas_call(
        body,
        grid=(n4 // blk4,),
        in_specs=[
            pl.BlockSpec((NC, blk4, CH), lambda i: (0, i, 0)),
            pl.BlockSpec((NC, blk4, CH), lambda i: (0, i, 0)),
            pl.BlockSpec((blk4, CH), lambda i: (i, 0)),
        ],
        out_specs=pl.BlockSpec((blk4, CH), lambda i: (i, 0)),
        out_shape=jax.ShapeDtypeStruct((n4, CH), F32),
    )(sum1p, cntp, r1p)


def _tc_post(sum2p, cntp, hp, w4l2, w4r2, b4out, blk4):
    n4 = hp.shape[0]
    dout4 = w4l2.shape[1]

    def body(s_ref, c_ref, h_ref, wl_ref, wr_ref, b_ref, o_ref):
        ssum = s_ref[0] + s_ref[1]
        cc = c_ref[0] + c_ref[1]
        mean2 = ssum * (1.0 / jnp.maximum(cc, 1.0))
        o = (jnp.dot(mean2, wl_ref[...], preferred_element_type=F32)
             + b_ref[...]
             + jnp.dot(h_ref[...], wr_ref[...], preferred_element_type=F32))
        o_ref[...] = jnp.maximum(o, 0.0)

    return pl.pallas_call(
        body,
        grid=(n4 // blk4,),
        in_specs=[
            pl.BlockSpec((NC, blk4, CH), lambda i: (0, i, 0)),
            pl.BlockSpec((NC, blk4, CH), lambda i: (0, i, 0)),
            pl.BlockSpec((blk4, CH), lambda i: (i, 0)),
            pl.BlockSpec((CH, dout4), lambda i: (0, 0)),
            pl.BlockSpec((CH, dout4), lambda i: (0, 0)),
            pl.BlockSpec((1, dout4), lambda i: (0, 0)),
        ],
        out_specs=pl.BlockSpec((blk4, dout4), lambda i: (i, 0)),
        out_shape=jax.ShapeDtypeStruct((n4, dout4), F32),
    )(sum2p, cntp, hp, w4l2, w4r2, b4out)


def kernel(x, edge_index, W1_l, W1_r, b1, W2_l, W2_r, b2):
    n, din = x.shape
    dh = W1_l.shape[1]
    dout = W2_l.shape[1]
    e = edge_index.shape[1]
    chunks = e // CH
    kw = -(-chunks // NW)
    rows_out = NW * kw

    e3 = edge_index.astype(jnp.int32).reshape(2, chunks, CH)
    src_p, dst_p = _tc_repack_edges(e3, n, rows_out)

    eye4 = jnp.eye(4, dtype=F32)
    x4 = x.reshape(n // 4, 4 * din)
    w4l = jnp.kron(eye4, W1_l)
    w4r = jnp.kron(eye4, W1_r)
    b4 = jnp.tile(b1, 4).reshape(1, CH)

    blk4 = n // 4
    y1p, r1p = _tc_pre(x4, w4l, w4r, b4, blk4)

    cnt = _sc_counts(dst_p, n)
    cntp = cnt.reshape(NC, n // 4, CH)
    sum1 = _sc_segment_sum(y1p.reshape(n, dh), src_p, dst_p, n)
    hp = _tc_mid(sum1.reshape(NC, n // 4, CH), cntp, r1p, blk4)

    sum2 = _sc_segment_sum(hp.reshape(n, dh), src_p, dst_p, n)
    w4l2 = jnp.kron(eye4, W2_l)
    w4r2 = jnp.kron(eye4, W2_r)
    b4out = jnp.tile(b2, 4).reshape(1, 4 * dout)
    outp = _tc_post(sum2.reshape(NC, n // 4, CH), cntp, hp,
                    w4l2, w4r2, b4out, blk4)
    return outp.reshape(n, dout)


# speedup vs baseline: 18.6965x; 1.0168x over previous
import functools

import jax
import jax.numpy as jnp
from jax import lax
from jax.experimental import pallas as pl
from jax.experimental.pallas import tpu as pltpu
from jax.experimental.pallas import tpu_sc as plsc

NC = 2
NS = 16
NW = NC * NS
CH = 128
F32 = jnp.float32


def _flat_worker_id():
    return lax.axis_index("s") * NC + lax.axis_index("c")


def _zero_zbuf(zbuf, zr, w):
    def body(i, carry):
        for c0 in range(0, w, 16):
            zbuf[i, c0:c0 + 16] = jnp.zeros((16,), F32)
        return carry
    lax.fori_loop(0, zr, body, 0)


def _zero_acc_async(zbuf, acc, base, rows, zr, zsem):
    n_full, rem = rows // zr, rows % zr
    for t in range(n_full):
        pltpu.async_copy(zbuf, acc.at[pl.ds(base + t * zr, zr)], zsem)
    if rem:
        pltpu.async_copy(zbuf.at[0:rem], acc.at[pl.ds(base + n_full * zr, rem)], zsem)
    for t in range(n_full):
        pltpu.make_async_copy(zbuf, acc.at[pl.ds(base + t * zr, zr)], zsem).wait()
    if rem:
        pltpu.make_async_copy(zbuf.at[0:rem], acc.at[pl.ds(base + n_full * zr, rem)], zsem).wait()


def _write_partial(acc, out_h, c, s, n):
    per_t = (((n + NS - 1) // NS) + 7) // 8 * 8
    last = n - (NS - 1) * per_t

    @pl.when(s < NS - 1)
    def _():
        base = s * per_t
        pltpu.sync_copy(acc.at[pl.ds(base, per_t)],
                        out_h.at[c].at[pl.ds(base, per_t)])

    @pl.when(s == NS - 1)
    def _():
        base = (NS - 1) * per_t
        pltpu.sync_copy(acc.at[pl.ds(base, last)],
                        out_h.at[c].at[pl.ds(base, last)])


def _idx_block(k, cap, mult=2):
    for d in range(cap, 1, -1):
        if d % mult == 0 and k % d == 0:
            return d
    return 0


def _tc_repack_edges(e3, n_nodes, rows_out):
    _, chunks, _ = e3.shape
    blk = rows_out // 8

    def body(e_ref, s_ref, d_ref):
        i = pl.program_id(0)
        row = lax.broadcasted_iota(jnp.int32, (blk, CH), 0) + i * blk
        valid = row < chunks
        s_ref[...] = jnp.where(valid, e_ref[0], 0)
        d_ref[...] = jnp.where(valid, e_ref[1], n_nodes)

    return pl.pallas_call(
        body,
        grid=(8,),
        in_specs=[pl.BlockSpec((2, blk, CH), lambda i: (0, i, 0))],
        out_specs=[
            pl.BlockSpec((blk, CH), lambda i: (i, 0)),
            pl.BlockSpec((blk, CH), lambda i: (i, 0)),
        ],
        out_shape=[
            jax.ShapeDtypeStruct((rows_out, CH), jnp.int32),
            jax.ShapeDtypeStruct((rows_out, CH), jnp.int32),
        ],
    )(e3)


def _sc_segment_sum(tab, src_p, dst_p, n_nodes):
    n, w = tab.shape
    kw = src_p.shape[0] // NW
    ib = _idx_block(kw, 28, mult=4)
    nb = kw // ib
    nq = ib // 4
    nacc = ((n + 1 + 127) // 128) * 128
    rows_per_tile_z = nacc // NS
    zr = 128
    mesh = plsc.VectorSubcoreMesh(core_axis_name="c", subcore_axis_name="s",
                                  num_cores=NC, num_subcores=NS)

    @functools.partial(
        pl.kernel,
        out_type=jax.ShapeDtypeStruct((NC, n, w), F32),
        mesh=mesh,
        compiler_params=pltpu.CompilerParams(use_tc_tiling_on_sc=False),
        scratch_types=[
            pltpu.VMEM((ib, CH), jnp.int32),
            pltpu.VMEM((ib, CH), jnp.int32),
            [pltpu.VMEM((CH, w), F32)] * 4,
            pltpu.VMEM((zr, w), F32),
            pltpu.VMEM_SHARED((nacc, w), F32),
            [pltpu.SemaphoreType.DMA] * 4,
            [pltpu.SemaphoreType.DMA] * 4,
            pltpu.SemaphoreType.DMA,
        ],
    )
    def k_fn(tab_h, src_h, dst_h, out_h, src_v, dst_v, rows, zbuf,
             acc, gs, ss, zsem):
        c = lax.axis_index("c")
        s = lax.axis_index("s")
        wid = _flat_worker_id()
        _zero_zbuf(zbuf, zr, w)
        _zero_acc_async(zbuf, acc, s * rows_per_tile_z, rows_per_tile_z, zr, zsem)
        plsc.subcore_barrier()

        def gath(j, r):
            pltpu.async_copy(tab_h.at[src_v.at[j]], rows[r], gs[r])

        def gath_wait(r):
            pltpu.make_async_copy(tab_h.at[src_v.at[0]], rows[r], gs[r]).wait()

        def scat(j, r):
            pltpu.async_copy(rows[r], acc.at[dst_v.at[j]], ss[r], add=True)

        def scat_wait(r):
            pltpu.make_async_copy(rows[r], acc.at[dst_v.at[0]], ss[r]).wait()

        def blk_body(b, carry):
            base = wid * kw + b * ib
            pltpu.sync_copy(src_h.at[pl.ds(base, ib)], src_v)
            pltpu.sync_copy(dst_h.at[pl.ds(base, ib)], dst_v)
            for r in range(3):
                gath(r, r)

            def quad(q, carry2):
                j = 4 * q
                gath_wait(0)
                scat(j, 0)
                gath_wait(1)

                @pl.when(q > 0)
                def _():
                    scat_wait(3)
                gath(j + 3, 3)
                scat(j + 1, 1)
                gath_wait(2)
                scat_wait(0)

                @pl.when(q < nq - 1)
                def _():
                    gath(j + 4, 0)
                scat(j + 2, 2)
                gath_wait(3)
                scat_wait(1)

                @pl.when(q < nq - 1)
                def _():
                    gath(j + 5, 1)
                scat(j + 3, 3)
                scat_wait(2)

                @pl.when(q < nq - 1)
                def _():
                    gath(j + 6, 2)
                return carry2
            lax.fori_loop(0, nq, quad, 0)
            scat_wait(3)
            return carry
        lax.fori_loop(0, nb, blk_body, 0)
        plsc.subcore_barrier()
        _write_partial(acc, out_h, c, s, n)

    return k_fn(tab, src_p, dst_p)


def _sc_counts(dst_p, n_nodes):
    n = n_nodes
    w = 32
    kw = dst_p.shape[0] // NW
    ib = _idx_block(kw, 16)
    nb = kw // ib
    nacc = ((n + 1 + 127) // 128) * 128
    rows_per_tile_z = nacc // NS
    zr = 128
    mesh = plsc.VectorSubcoreMesh(core_axis_name="c", subcore_axis_name="s",
                                  num_cores=NC, num_subcores=NS)

    @functools.partial(
        pl.kernel,
        out_type=jax.ShapeDtypeStruct((NC, n, w), F32),
        mesh=mesh,
        compiler_params=pltpu.CompilerParams(use_tc_tiling_on_sc=False),
        scratch_types=[
            pltpu.VMEM((ib, CH), jnp.int32),
            pltpu.VMEM((CH, w), F32),
            pltpu.VMEM((zr, w), F32),
            pltpu.VMEM_SHARED((nacc, w), F32),
            pltpu.SemaphoreType.DMA,
            pltpu.SemaphoreType.DMA,
        ],
    )
    def k_fn(dst_h, out_h, dst_v, ones_v, zbuf, acc, sem, zsem):
        c = lax.axis_index("c")
        s = lax.axis_index("s")
        wid = _flat_worker_id()

        def fill_ones(i, carry):
            for c0 in range(0, w, 16):
                ones_v[i, c0:c0 + 16] = jnp.ones((16,), F32)
            return carry
        lax.fori_loop(0, CH, fill_ones, 0)
        _zero_zbuf(zbuf, zr, w)
        _zero_acc_async(zbuf, acc, s * rows_per_tile_z, rows_per_tile_z, zr, zsem)
        plsc.subcore_barrier()

        def blk_body(b, carry):
            base = wid * kw + b * ib
            pltpu.sync_copy(dst_h.at[pl.ds(base, ib)], dst_v)

            def body(j, carry2):
                pltpu.async_copy(ones_v, acc.at[dst_v.at[j]], sem, add=True)
                return carry2
            lax.fori_loop(0, ib, body, 0)
            for _ in range(ib):
                pltpu.make_async_copy(ones_v, acc.at[dst_v.at[0]], sem).wait()
            return carry
        lax.fori_loop(0, nb, blk_body, 0)
        plsc.subcore_barrier()
        _write_partial(acc, out_h, c, s, n)

    return k_fn(dst_p)


def _tc_pre(x4, w4l, w4r, b4, blk4):
    n4, din4 = x4.shape

    def body(x_ref, wl_ref, wr_ref, b_ref, y_ref, r_ref):
        xb = x_ref[...]
        y_ref[...] = jnp.dot(xb, wl_ref[...], preferred_element_type=F32)
        r_ref[...] = jnp.dot(xb, wr_ref[...], preferred_element_type=F32) + b_ref[...]

    return pl.pallas_call(
        body,
        grid=(n4 // blk4,),
        in_specs=[
            pl.BlockSpec((blk4, din4), lambda i: (i, 0)),
            pl.BlockSpec((din4, CH), lambda i: (0, 0)),
            pl.BlockSpec((din4, CH), lambda i: (0, 0)),
            pl.BlockSpec((1, CH), lambda i: (0, 0)),
        ],
        out_specs=[
            pl.BlockSpec((blk4, CH), lambda i: (i, 0)),
            pl.BlockSpec((blk4, CH), lambda i: (i, 0)),
        ],
        out_shape=[
            jax.ShapeDtypeStruct((n4, CH), F32),
            jax.ShapeDtypeStruct((n4, CH), F32),
        ],
    )(x4, w4l, w4r, b4)


def _tc_mid(sum1p, cntp, r1p, blk4):
    n4 = r1p.shape[0]

    def body(s_ref, c_ref, r_ref, h_ref):
        ssum = s_ref[0] + s_ref[1]
        cc = c_ref[0] + c_ref[1]
        rcp = 1.0 / jnp.maximum(cc, 1.0)
        h_ref[...] = jnp.maximum(ssum * rcp + r_ref[...], 0.0)

    return pl.pallas_call(
        body,
        grid=(n4 // blk4,),
        in_specs=[
            pl.BlockSpec((NC, blk4, CH), lambda i: (0, i, 0)),
            pl.BlockSpec((NC, blk4, CH), lambda i: (0, i, 0)),
            pl.BlockSpec((blk4, CH), lambda i: (i, 0)),
        ],
        out_specs=pl.BlockSpec((blk4, CH), lambda i: (i, 0)),
        out_shape=jax.ShapeDtypeStruct((n4, CH), F32),
    )(sum1p, cntp, r1p)


def _tc_post(sum2p, cntp, hp, w4l2, w4r2, b4out, blk4):
    n4 = hp.shape[0]
    dout4 = w4l2.shape[1]

    def body(s_ref, c_ref, h_ref, wl_ref, wr_ref, b_ref, o_ref):
        ssum = s_ref[0] + s_ref[1]
        cc = c_ref[0] + c_ref[1]
        mean2 = ssum * (1.0 / jnp.maximum(cc, 1.0))
        o = (jnp.dot(mean2, wl_ref[...], preferred_element_type=F32)
             + b_ref[...]
             + jnp.dot(h_ref[...], wr_ref[...], preferred_element_type=F32))
        o_ref[...] = jnp.maximum(o, 0.0)

    return pl.pallas_call(
        body,
        grid=(n4 // blk4,),
        in_specs=[
            pl.BlockSpec((NC, blk4, CH), lambda i: (0, i, 0)),
            pl.BlockSpec((NC, blk4, CH), lambda i: (0, i, 0)),
            pl.BlockSpec((blk4, CH), lambda i: (i, 0)),
            pl.BlockSpec((CH, dout4), lambda i: (0, 0)),
            pl.BlockSpec((CH, dout4), lambda i: (0, 0)),
            pl.BlockSpec((1, dout4), lambda i: (0, 0)),
        ],
        out_specs=pl.BlockSpec((blk4, dout4), lambda i: (i, 0)),
        out_shape=jax.ShapeDtypeStruct((n4, dout4), F32),
    )(sum2p, cntp, hp, w4l2, w4r2, b4out)


def kernel(x, edge_index, W1_l, W1_r, b1, W2_l, W2_r, b2):
    n, din = x.shape
    dh = W1_l.shape[1]
    dout = W2_l.shape[1]
    e = edge_index.shape[1]
    chunks = e // CH
    kw = -(-chunks // NW)
    rows_out = NW * kw

    e3 = edge_index.astype(jnp.int32).reshape(2, chunks, CH)
    src_p, dst_p = _tc_repack_edges(e3, n, rows_out)

    eye4 = jnp.eye(4, dtype=F32)
    x4 = x.reshape(n // 4, 4 * din)
    w4l = jnp.kron(eye4, W1_l)
    w4r = jnp.kron(eye4, W1_r)
    b4 = jnp.tile(b1, 4).reshape(1, CH)

    blk4 = n // 4
    y1p, r1p = _tc_pre(x4, w4l, w4r, b4, blk4)

    cnt = _sc_counts(dst_p, n)
    cntp = cnt.reshape(NC, n // 4, CH)
    sum1 = _sc_segment_sum(y1p.reshape(n, dh), src_p, dst_p, n)
    hp = _tc_mid(sum1.reshape(NC, n // 4, CH), cntp, r1p, blk4)

    sum2 = _sc_segment_sum(hp.reshape(n, dh), src_p, dst_p, n)
    w4l2 = jnp.kron(eye4, W2_l)
    w4r2 = jnp.kron(eye4, W2_r)
    b4out = jnp.tile(b2, 4).reshape(1, 4 * dout)
    outp = _tc_post(sum2.reshape(NC, n // 4, CH), cntp, hp,
                    w4l2, w4r2, b4out, blk4)
    return outp.reshape(n, dout)


# speedup vs baseline: 20.0375x; 1.0717x over previous
import functools

import jax
import jax.numpy as jnp
from jax import lax
from jax.experimental import pallas as pl
from jax.experimental.pallas import tpu as pltpu
from jax.experimental.pallas import tpu_sc as plsc

NC = 2
NS = 16
NW = NC * NS
CH = 128
NBUF = 5
LOOKAHEAD = 3
F32 = jnp.float32


def _flat_worker_id():
    return lax.axis_index("s") * NC + lax.axis_index("c")


def _zero_zbuf(zbuf, zr, w):
    def body(i, carry):
        for c0 in range(0, w, 16):
            zbuf[i, c0:c0 + 16] = jnp.zeros((16,), F32)
        return carry
    lax.fori_loop(0, zr, body, 0)


def _zero_acc_async(zbuf, acc, base, rows, zr, zsem):
    n_full, rem = rows // zr, rows % zr
    for t in range(n_full):
        pltpu.async_copy(zbuf, acc.at[pl.ds(base + t * zr, zr)], zsem)
    if rem:
        pltpu.async_copy(zbuf.at[0:rem], acc.at[pl.ds(base + n_full * zr, rem)], zsem)
    for t in range(n_full):
        pltpu.make_async_copy(zbuf, acc.at[pl.ds(base + t * zr, zr)], zsem).wait()
    if rem:
        pltpu.make_async_copy(zbuf.at[0:rem], acc.at[pl.ds(base + n_full * zr, rem)], zsem).wait()


def _write_partial(acc, out_h, c, s, n):
    per_t = (((n + NS - 1) // NS) + 7) // 8 * 8
    last = n - (NS - 1) * per_t

    @pl.when(s < NS - 1)
    def _():
        base = s * per_t
        pltpu.sync_copy(acc.at[pl.ds(base, per_t)],
                        out_h.at[c].at[pl.ds(base, per_t)])

    @pl.when(s == NS - 1)
    def _():
        base = (NS - 1) * per_t
        pltpu.sync_copy(acc.at[pl.ds(base, last)],
                        out_h.at[c].at[pl.ds(base, last)])


def _idx_block(k, cap, mult=2):
    for d in range(cap, 1, -1):
        if d % mult == 0 and k % d == 0:
            return d
    return 0


def _tc_repack_edges(e3, n_nodes, rows_out):
    _, chunks, _ = e3.shape
    blk = rows_out // 8

    def body(e_ref, s_ref, d_ref):
        i = pl.program_id(0)
        row = lax.broadcasted_iota(jnp.int32, (blk, CH), 0) + i * blk
        valid = row < chunks
        s_ref[...] = jnp.where(valid, e_ref[0], 0)
        d_ref[...] = jnp.where(valid, e_ref[1], n_nodes)

    return pl.pallas_call(
        body,
        grid=(8,),
        in_specs=[pl.BlockSpec((2, blk, CH), lambda i: (0, i, 0))],
        out_specs=[
            pl.BlockSpec((blk, CH), lambda i: (i, 0)),
            pl.BlockSpec((blk, CH), lambda i: (i, 0)),
        ],
        out_shape=[
            jax.ShapeDtypeStruct((rows_out, CH), jnp.int32),
            jax.ShapeDtypeStruct((rows_out, CH), jnp.int32),
        ],
    )(e3)


def _sc_segment_sum(tab, src_p, dst_p, n_nodes):
    n, w = tab.shape
    kw = src_p.shape[0] // NW
    ib = _idx_block(kw, 28, mult=4)
    nb = kw // ib
    nacc = ((n + 1 + 127) // 128) * 128
    rows_per_tile_z = nacc // NS
    zr = 64
    mesh = plsc.VectorSubcoreMesh(core_axis_name="c", subcore_axis_name="s",
                                  num_cores=NC, num_subcores=NS)

    @functools.partial(
        pl.kernel,
        out_type=jax.ShapeDtypeStruct((NC, n, w), F32),
        mesh=mesh,
        compiler_params=pltpu.CompilerParams(use_tc_tiling_on_sc=False),
        scratch_types=[
            pltpu.VMEM((ib, CH), jnp.int32),
            pltpu.VMEM((ib, CH), jnp.int32),
            [pltpu.VMEM((CH, w), F32)] * NBUF,
            pltpu.VMEM((zr, w), F32),
            pltpu.VMEM_SHARED((nacc, w), F32),
            [pltpu.SemaphoreType.DMA] * NBUF,
            [pltpu.SemaphoreType.DMA] * NBUF,
            pltpu.SemaphoreType.DMA,
        ],
    )
    def k_fn(tab_h, src_h, dst_h, out_h, src_v, dst_v, rows, zbuf,
             acc, gs, ss, zsem):
        c = lax.axis_index("c")
        s = lax.axis_index("s")
        wid = _flat_worker_id()
        _zero_zbuf(zbuf, zr, w)
        _zero_acc_async(zbuf, acc, s * rows_per_tile_z, rows_per_tile_z, zr, zsem)
        plsc.subcore_barrier()

        def gath(j, r):
            pltpu.async_copy(tab_h.at[src_v.at[j]], rows[r], gs[r])

        def gath_wait(r):
            pltpu.make_async_copy(tab_h.at[src_v.at[0]], rows[r], gs[r]).wait()

        def scat(j, r):
            pltpu.async_copy(rows[r], acc.at[dst_v.at[j]], ss[r], add=True)

        def scat_wait(r):
            pltpu.make_async_copy(rows[r], acc.at[dst_v.at[0]], ss[r]).wait()

        def blk_body(b, carry):
            base = wid * kw + b * ib
            pltpu.sync_copy(src_h.at[pl.ds(base, ib)], src_v)
            pltpu.sync_copy(dst_h.at[pl.ds(base, ib)], dst_v)
            for j in range(LOOKAHEAD):
                gath(j, j % NBUF)
            for j in range(ib):
                r = j % NBUF
                gath_wait(r)
                scat(j, r)
                jn = j + LOOKAHEAD
                if jn < ib:
                    rn = jn % NBUF
                    if jn >= NBUF:
                        scat_wait(rn)
                    gath(jn, rn)
            for j in range(ib - NBUF, ib):
                scat_wait(j % NBUF)
            return carry
        lax.fori_loop(0, nb, blk_body, 0)
        plsc.subcore_barrier()
        _write_partial(acc, out_h, c, s, n)

    return k_fn(tab, src_p, dst_p)


def _sc_counts(dst_p, n_nodes):
    n = n_nodes
    w = 32
    kw = dst_p.shape[0] // NW
    ib = _idx_block(kw, 16)
    nb = kw // ib
    nacc = ((n + 1 + 127) // 128) * 128
    rows_per_tile_z = nacc // NS
    zr = 128
    mesh = plsc.VectorSubcoreMesh(core_axis_name="c", subcore_axis_name="s",
                                  num_cores=NC, num_subcores=NS)

    @functools.partial(
        pl.kernel,
        out_type=jax.ShapeDtypeStruct((NC, n, w), F32),
        mesh=mesh,
        compiler_params=pltpu.CompilerParams(use_tc_tiling_on_sc=False),
        scratch_types=[
            pltpu.VMEM((ib, CH), jnp.int32),
            pltpu.VMEM((CH, w), F32),
            pltpu.VMEM((zr, w), F32),
            pltpu.VMEM_SHARED((nacc, w), F32),
            pltpu.SemaphoreType.DMA,
            pltpu.SemaphoreType.DMA,
        ],
    )
    def k_fn(dst_h, out_h, dst_v, ones_v, zbuf, acc, sem, zsem):
        c = lax.axis_index("c")
        s = lax.axis_index("s")
        wid = _flat_worker_id()

        def fill_ones(i, carry):
            for c0 in range(0, w, 16):
                ones_v[i, c0:c0 + 16] = jnp.ones((16,), F32)
            return carry
        lax.fori_loop(0, CH, fill_ones, 0)
        _zero_zbuf(zbuf, zr, w)
        _zero_acc_async(zbuf, acc, s * rows_per_tile_z, rows_per_tile_z, zr, zsem)
        plsc.subcore_barrier()

        def blk_body(b, carry):
            base = wid * kw + b * ib
            pltpu.sync_copy(dst_h.at[pl.ds(base, ib)], dst_v)

            def body(j, carry2):
                pltpu.async_copy(ones_v, acc.at[dst_v.at[j]], sem, add=True)
                return carry2
            lax.fori_loop(0, ib, body, 0)
            for _ in range(ib):
                pltpu.make_async_copy(ones_v, acc.at[dst_v.at[0]], sem).wait()
            return carry
        lax.fori_loop(0, nb, blk_body, 0)
        plsc.subcore_barrier()
        _write_partial(acc, out_h, c, s, n)

    return k_fn(dst_p)


def _tc_pre(x4, w4l, w4r, b4, blk4):
    n4, din4 = x4.shape

    def body(x_ref, wl_ref, wr_ref, b_ref, y_ref, r_ref):
        xb = x_ref[...]
        y_ref[...] = jnp.dot(xb, wl_ref[...], preferred_element_type=F32)
        r_ref[...] = jnp.dot(xb, wr_ref[...], preferred_element_type=F32) + b_ref[...]

    return pl.pallas_call(
        body,
        grid=(n4 // blk4,),
        in_specs=[
            pl.BlockSpec((blk4, din4), lambda i: (i, 0)),
            pl.BlockSpec((din4, CH), lambda i: (0, 0)),
            pl.BlockSpec((din4, CH), lambda i: (0, 0)),
            pl.BlockSpec((1, CH), lambda i: (0, 0)),
        ],
        out_specs=[
            pl.BlockSpec((blk4, CH), lambda i: (i, 0)),
            pl.BlockSpec((blk4, CH), lambda i: (i, 0)),
        ],
        out_shape=[
            jax.ShapeDtypeStruct((n4, CH), F32),
            jax.ShapeDtypeStruct((n4, CH), F32),
        ],
    )(x4, w4l, w4r, b4)


def _tc_mid(sum1p, cntp, r1p, blk4):
    n4 = r1p.shape[0]

    def body(s_ref, c_ref, r_ref, h_ref):
        ssum = s_ref[0] + s_ref[1]
        cc = c_ref[0] + c_ref[1]
        rcp = 1.0 / jnp.maximum(cc, 1.0)
        h_ref[...] = jnp.maximum(ssum * rcp + r_ref[...], 0.0)

    return pl.pallas_call(
        body,
        grid=(n4 // blk4,),
        in_specs=[
            pl.BlockSpec((NC, blk4, CH), lambda i: (0, i, 0)),
            pl.BlockSpec((NC, blk4, CH), lambda i: (0, i, 0)),
            pl.BlockSpec((blk4, CH), lambda i: (i, 0)),
        ],
        out_specs=pl.BlockSpec((blk4, CH), lambda i: (i, 0)),
        out_shape=jax.ShapeDtypeStruct((n4, CH), F32),
    )(sum1p, cntp, r1p)


def _tc_post(sum2p, cntp, hp, w4l2, w4r2, b4out, blk4):
    n4 = hp.shape[0]
    dout4 = w4l2.shape[1]

    def body(s_ref, c_ref, h_ref, wl_ref, wr_ref, b_ref, o_ref):
        ssum = s_ref[0] + s_ref[1]
        cc = c_ref[0] + c_ref[1]
        mean2 = ssum * (1.0 / jnp.maximum(cc, 1.0))
        o = (jnp.dot(mean2, wl_ref[...], preferred_element_type=F32)
             + b_ref[...]
             + jnp.dot(h_ref[...], wr_ref[...], preferred_element_type=F32))
        o_ref[...] = jnp.maximum(o, 0.0)

    return pl.pallas_call(
        body,
        grid=(n4 // blk4,),
        in_specs=[
            pl.BlockSpec((NC, blk4, CH), lambda i: (0, i, 0)),
            pl.BlockSpec((NC, blk4, CH), lambda i: (0, i, 0)),
            pl.BlockSpec((blk4, CH), lambda i: (i, 0)),
            pl.BlockSpec((CH, dout4), lambda i: (0, 0)),
            pl.BlockSpec((CH, dout4), lambda i: (0, 0)),
            pl.BlockSpec((1, dout4), lambda i: (0, 0)),
        ],
        out_specs=pl.BlockSpec((blk4, dout4), lambda i: (i, 0)),
        out_shape=jax.ShapeDtypeStruct((n4, dout4), F32),
    )(sum2p, cntp, hp, w4l2, w4r2, b4out)


def kernel(x, edge_index, W1_l, W1_r, b1, W2_l, W2_r, b2):
    n, din = x.shape
    dh = W1_l.shape[1]
    dout = W2_l.shape[1]
    e = edge_index.shape[1]
    chunks = e // CH
    kw = -(-chunks // NW)
    rows_out = NW * kw

    e3 = edge_index.astype(jnp.int32).reshape(2, chunks, CH)
    src_p, dst_p = _tc_repack_edges(e3, n, rows_out)

    eye4 = jnp.eye(4, dtype=F32)
    x4 = x.reshape(n // 4, 4 * din)
    w4l = jnp.kron(eye4, W1_l)
    w4r = jnp.kron(eye4, W1_r)
    b4 = jnp.tile(b1, 4).reshape(1, CH)

    blk4 = n // 4
    y1p, r1p = _tc_pre(x4, w4l, w4r, b4, blk4)

    cnt = _sc_counts(dst_p, n)
    cntp = cnt.reshape(NC, n // 4, CH)
    sum1 = _sc_segment_sum(y1p.reshape(n, dh), src_p, dst_p, n)
    hp = _tc_mid(sum1.reshape(NC, n // 4, CH), cntp, r1p, blk4)

    sum2 = _sc_segment_sum(hp.reshape(n, dh), src_p, dst_p, n)
    w4l2 = jnp.kron(eye4, W2_l)
    w4r2 = jnp.kron(eye4, W2_r)
    b4out = jnp.tile(b2, 4).reshape(1, 4 * dout)
    outp = _tc_post(sum2.reshape(NC, n // 4, CH), cntp, hp,
                    w4l2, w4r2, b4out, blk4)
    return outp.reshape(n, dout)
